# baseline (speedup 1.0000x reference)
import jax
import jax.numpy as jnp
from jax import lax
from jax.experimental import pallas as pl
from jax.experimental.pallas import tpu as pltpu

_B = 16384
_D = 256
_R = 1024
_NB = _B // _R


def _gelu_exact(x):
    return 0.5 * x * (1.0 + lax.erf(x * (2.0 ** -0.5)))


def _body(gidx_ref, aidx_ref, xt_ref, cls_ref, gtab_ref, atab_ref, bmb_ref,
          w1_ref, b1_ref, w2_ref, b2_ref, w3_ref, b3_ref, out_ref,
          buf_ref, sem_ref):
    i = pl.program_id(0)
    s = lax.rem(i, 2)
    r = _R

    def copies(step, slot):
        return pltpu.make_async_copy(
            buf_ref.at[lax.rem(step, 2), slot],
            out_ref.at[pl.ds(step * _R, _R), slot, :],
            sem_ref.at[lax.rem(step, 2), slot])

    @pl.when(i >= 2)
    def _():
        for j in range(4):
            copies(i - 2, j).wait()

    buf_ref[s, 0] = jnp.broadcast_to(cls_ref[...], (r, _D))
    copies(i, 0).start()

    g = gidx_ref[0, 0, :]
    goh = (g[:, None] == lax.broadcasted_iota(jnp.int32, (r, 2), 1)
           ).astype(jnp.float32)
    buf_ref[s, 1] = jnp.dot(goh, gtab_ref[...],
                            preferred_element_type=jnp.float32)
    copies(i, 1).start()

    a = aidx_ref[0, 0, :]
    aoh = (a[:, None] == lax.broadcasted_iota(jnp.int32, (r, 7), 1)
           ).astype(jnp.float32)
    buf_ref[s, 2] = jnp.dot(aoh, atab_ref[...],
                            preferred_element_type=jnp.float32)
    copies(i, 2).start()

    h = lax.dot_general(xt_ref[...], w1_ref[...], (((0,), (0,)), ((), ())),
                        preferred_element_type=jnp.float32) + b1_ref[...]
    h = _gelu_exact(h)
    h = jnp.dot(h, w2_ref[...], preferred_element_type=jnp.float32) + b2_ref[...]
    h = _gelu_exact(h)
    h = jnp.dot(h, w3_ref[...], preferred_element_type=jnp.float32) + b3_ref[...]
    buf_ref[s, 3] = h + bmb_ref[...]
    copies(i, 3).start()

    @pl.when(i == _NB - 1)
    def _():
        for j in range(4):
            copies(i - 1, j).wait()
        for j in range(4):
            copies(i, j).wait()


def kernel(user_gender, user_age_bin, user_born_mort, cls_param, gender_table,
           age_table, born_mort_bias, W1, b1, W2, b2, W3, b3):
    n = user_born_mort.shape[0]
    gidx = user_gender.astype(jnp.int32).reshape(_NB, 1, _R)
    aidx = user_age_bin.astype(jnp.int32).reshape(_NB, 1, _R)
    xt = user_born_mort.T

    full = lambda shape: pl.BlockSpec(shape, lambda i: (0,) * len(shape))
    out3d = pl.pallas_call(
        _body,
        grid=(_NB,),
        in_specs=[
            pl.BlockSpec((1, 1, _R), lambda i: (i, 0, 0)),
            pl.BlockSpec((1, 1, _R), lambda i: (i, 0, 0)),
            pl.BlockSpec((13, _R), lambda i: (0, i)),
            full((1, _D)),
            full((2, _D)),
            full((7, _D)),
            full((1, _D)),
            full((13, 64)),
            full((1, 64)),
            full((64, 128)),
            full((1, 128)),
            full((128, _D)),
            full((1, _D)),
        ],
        out_specs=pl.BlockSpec(memory_space=pl.ANY),
        out_shape=jax.ShapeDtypeStruct((n, 4, _D), jnp.float32),
        scratch_shapes=[
            pltpu.VMEM((2, 4, _R, _D), jnp.float32),
            pltpu.SemaphoreType.DMA((2, 4)),
        ],
        compiler_params=pltpu.CompilerParams(
            dimension_semantics=("arbitrary",)),
    )(gidx, aidx, xt, cls_param, gender_table, age_table,
      born_mort_bias, W1, b1.reshape(1, 64), W2, b2.reshape(1, 128),
      W3, b3.reshape(1, _D))

    mask = jnp.ones((n, 4), dtype=jnp.int32)
    return (out3d, mask)


# speedup vs baseline: 1.2230x; 1.2230x over previous
import jax
import jax.numpy as jnp
from jax import lax
from jax.experimental import pallas as pl
from jax.experimental.pallas import tpu as pltpu

_B = 16384
_D = 256
_R = 4096
_NB = _B // _R


def _gelu_exact(x):
    return 0.5 * x * (1.0 + lax.erf(x * (2.0 ** -0.5)))


def _body(gidx_ref, aidx_ref, xt_ref, cls_ref, gtab_ref, atab_ref, bmb_ref,
          w1_ref, b1_ref, w2_ref, b2_ref, w3_ref, b3_ref, out_ref,
          buf_ref, sem_ref):
    i = pl.program_id(0)
    s = lax.rem(i, 2)
    r = _R

    def copies(step, slot):
        return pltpu.make_async_copy(
            buf_ref.at[lax.rem(step, 2), slot],
            out_ref.at[pl.ds(step * _R, _R), slot, :],
            sem_ref.at[lax.rem(step, 2), slot])

    @pl.when(i >= 2)
    def _():
        for j in range(4):
            copies(i - 2, j).wait()

    buf_ref[s, 0] = jnp.broadcast_to(cls_ref[...], (r, _D))
    copies(i, 0).start()

    g = gidx_ref[0, 0, :]
    goh = (g[:, None] == lax.broadcasted_iota(jnp.int32, (r, 2), 1)
           ).astype(jnp.float32)
    buf_ref[s, 1] = jnp.dot(goh, gtab_ref[...],
                            preferred_element_type=jnp.float32)
    copies(i, 1).start()

    a = aidx_ref[0, 0, :]
    aoh = (a[:, None] == lax.broadcasted_iota(jnp.int32, (r, 7), 1)
           ).astype(jnp.float32)
    buf_ref[s, 2] = jnp.dot(aoh, atab_ref[...],
                            preferred_element_type=jnp.float32)
    copies(i, 2).start()

    h = lax.dot_general(xt_ref[...], w1_ref[...], (((0,), (0,)), ((), ())),
                        preferred_element_type=jnp.float32) + b1_ref[...]
    h = _gelu_exact(h)
    h = jnp.dot(h, w2_ref[...], preferred_element_type=jnp.float32) + b2_ref[...]
    h = _gelu_exact(h)
    h = jnp.dot(h, w3_ref[...], preferred_element_type=jnp.float32) + b3_ref[...]
    buf_ref[s, 3] = h + bmb_ref[...]
    copies(i, 3).start()

    @pl.when(i == _NB - 1)
    def _():
        for j in range(4):
            copies(i - 1, j).wait()
        for j in range(4):
            copies(i, j).wait()


def kernel(user_gender, user_age_bin, user_born_mort, cls_param, gender_table,
           age_table, born_mort_bias, W1, b1, W2, b2, W3, b3):
    n = user_born_mort.shape[0]
    gidx = user_gender.astype(jnp.int32).reshape(_NB, 1, _R)
    aidx = user_age_bin.astype(jnp.int32).reshape(_NB, 1, _R)
    xt = user_born_mort.T

    full = lambda shape: pl.BlockSpec(shape, lambda i: (0,) * len(shape))
    out3d = pl.pallas_call(
        _body,
        grid=(_NB,),
        in_specs=[
            pl.BlockSpec((1, 1, _R), lambda i: (i, 0, 0)),
            pl.BlockSpec((1, 1, _R), lambda i: (i, 0, 0)),
            pl.BlockSpec((13, _R), lambda i: (0, i)),
            full((1, _D)),
            full((2, _D)),
            full((7, _D)),
            full((1, _D)),
            full((13, 64)),
            full((1, 64)),
            full((64, 128)),
            full((1, 128)),
            full((128, _D)),
            full((1, _D)),
        ],
        out_specs=pl.BlockSpec(memory_space=pl.ANY),
        out_shape=jax.ShapeDtypeStruct((n, 4, _D), jnp.float32),
        scratch_shapes=[
            pltpu.VMEM((2, 4, _R, _D), jnp.float32),
            pltpu.SemaphoreType.DMA((2, 4)),
        ],
        compiler_params=pltpu.CompilerParams(
            dimension_semantics=("arbitrary",)),
    )(gidx, aidx, xt, cls_param, gender_table, age_table,
      born_mort_bias, W1, b1.reshape(1, 64), W2, b2.reshape(1, 128),
      W3, b3.reshape(1, _D))

    mask = jnp.ones((n, 4), dtype=jnp.int32)
    return (out3d, mask)


# speedup vs baseline: 1.2738x; 1.0416x over previous
import jax
import jax.numpy as jnp
from jax import lax
from jax.experimental import pallas as pl
from jax.experimental.pallas import tpu as pltpu

_B = 16384
_D = 256
_R = 2048
_NB = _B // _R


def _gelu_exact(x):
    return 0.5 * x * (1.0 + lax.erf(x * (2.0 ** -0.5)))


def _body(gidx_ref, aidx_ref, xt_ref, cls_ref, gtab_ref, atab_ref, bmb_ref,
          w1_ref, b1_ref, w2_ref, b2_ref, w3_ref, b3_ref, out_ref,
          buf_ref, sem_ref):
    i = pl.program_id(0)
    s = lax.rem(i, 2)
    r = _R

    def copies(step, slot):
        return pltpu.make_async_copy(
            buf_ref.at[lax.rem(step, 2), slot],
            out_ref.at[pl.ds(step * _R, _R), slot, :],
            sem_ref.at[lax.rem(step, 2), slot])

    @pl.when(i >= 2)
    def _():
        for j in range(4):
            copies(i - 2, j).wait()

    buf_ref[s, 0] = jnp.broadcast_to(cls_ref[...], (r, _D))
    copies(i, 0).start()

    g = gidx_ref[0, 0, :]
    goh = (g[:, None] == lax.broadcasted_iota(jnp.int32, (r, 2), 1)
           ).astype(jnp.float32)
    buf_ref[s, 1] = jnp.dot(goh, gtab_ref[...],
                            preferred_element_type=jnp.float32)
    copies(i, 1).start()

    a = aidx_ref[0, 0, :]
    aoh = (a[:, None] == lax.broadcasted_iota(jnp.int32, (r, 7), 1)
           ).astype(jnp.float32)
    buf_ref[s, 2] = jnp.dot(aoh, atab_ref[...],
                            preferred_element_type=jnp.float32)
    copies(i, 2).start()

    h = lax.dot_general(xt_ref[...], w1_ref[...], (((0,), (0,)), ((), ())),
                        preferred_element_type=jnp.float32) + b1_ref[...]
    h = _gelu_exact(h)
    h = jnp.dot(h, w2_ref[...], preferred_element_type=jnp.float32) + b2_ref[...]
    h = _gelu_exact(h)
    h = jnp.dot(h, w3_ref[...], preferred_element_type=jnp.float32) + b3_ref[...]
    buf_ref[s, 3] = h + bmb_ref[...]
    copies(i, 3).start()

    @pl.when(i == _NB - 1)
    def _():
        for j in range(4):
            copies(i - 1, j).wait()
        for j in range(4):
            copies(i, j).wait()


def kernel(user_gender, user_age_bin, user_born_mort, cls_param, gender_table,
           age_table, born_mort_bias, W1, b1, W2, b2, W3, b3):
    n = user_born_mort.shape[0]
    gidx = user_gender.astype(jnp.int32).reshape(_NB, 1, _R)
    aidx = user_age_bin.astype(jnp.int32).reshape(_NB, 1, _R)
    xt = user_born_mort.T

    full = lambda shape: pl.BlockSpec(shape, lambda i: (0,) * len(shape))
    out3d = pl.pallas_call(
        _body,
        grid=(_NB,),
        in_specs=[
            pl.BlockSpec((1, 1, _R), lambda i: (i, 0, 0)),
            pl.BlockSpec((1, 1, _R), lambda i: (i, 0, 0)),
            pl.BlockSpec((13, _R), lambda i: (0, i)),
            full((1, _D)),
            full((2, _D)),
            full((7, _D)),
            full((1, _D)),
            full((13, 64)),
            full((1, 64)),
            full((64, 128)),
            full((1, 128)),
            full((128, _D)),
            full((1, _D)),
        ],
        out_specs=pl.BlockSpec(memory_space=pl.ANY),
        out_shape=jax.ShapeDtypeStruct((n, 4, _D), jnp.float32),
        scratch_shapes=[
            pltpu.VMEM((2, 4, _R, _D), jnp.float32),
            pltpu.SemaphoreType.DMA((2, 4)),
        ],
        compiler_params=pltpu.CompilerParams(
            dimension_semantics=("arbitrary",)),
    )(gidx, aidx, xt, cls_param, gender_table, age_table,
      born_mort_bias, W1, b1.reshape(1, 64), W2, b2.reshape(1, 128),
      W3, b3.reshape(1, _D))

    mask = jnp.ones((n, 4), dtype=jnp.int32)
    return (out3d, mask)
